# baseline (speedup 1.0000x reference)
import dataclasses
import functools

import jax
import jax.numpy as jnp
from jax import lax
from jax.experimental import pallas as pl
from jax.experimental.pallas import tpu as pltpu
from jax.experimental.pallas import tpu_sc as plsc

N = 1569
NUM_REL = 10938
NH = 16
HPT = 2
NGRP = NH // HPT
ROWS_PER_TILE = 392
R = 8
NCHUNK = 49
NVREG = 99
TAIL_OFF = N - 16
LAST_ROW = N - 1


def _sc_bias_kernel(table_t, idx):
    mesh = plsc.VectorSubcoreMesh(core_axis_name="c", subcore_axis_name="s")

    blk_f32 = pltpu.VMEM((R, N), jnp.float32)
    blk_i32 = pltpu.VMEM((R, N), jnp.int32)
    row_f32 = pltpu.VMEM((1, N), jnp.float32)

    cp = pltpu.CompilerParams()
    if "needs_layout_passes" in pltpu.CompilerParams.__dataclass_fields__:
        cp = dataclasses.replace(cp, needs_layout_passes=False)

    @functools.partial(
        pl.kernel,
        out_type=jax.ShapeDtypeStruct((NH, N, N), jnp.float32),
        mesh=mesh,
        scratch_types=[
            pltpu.VMEM((HPT, NUM_REL), jnp.float32),
            (blk_i32, blk_i32),
            (tuple(blk_f32 for _ in range(HPT)),
             tuple(blk_f32 for _ in range(HPT))),
            pltpu.VMEM((1, N), jnp.int32),
            tuple(row_f32 for _ in range(HPT)),
            (pltpu.SemaphoreType.DMA, pltpu.SemaphoreType.DMA),
            (pltpu.SemaphoreType.DMA, pltpu.SemaphoreType.DMA),
        ],
        compiler_params=cp,
    )
    def kern(tab_hbm, idx_hbm, out_hbm, tab_v, idx_v, out_v, row_i, row_o,
             sem_i, sem_o):
        c = lax.axis_index("c")
        s = lax.axis_index("s")
        g = s % NGRP
        t = 2 * (s // NGRP) + c
        head_base = HPT * g
        row_base = t * ROWS_PER_TILE

        pltpu.sync_copy(tab_hbm.at[pl.ds(head_base, HPT)], tab_v)

        def chunk_start(ci):
            return pl.multiple_of(row_base + ci * R, R)

        pltpu.async_copy(idx_hbm.at[pl.ds(chunk_start(0), R)],
                         idx_v[0], sem_i[0])

        hvecs = [jnp.full((16,), hh, dtype=jnp.int32) for hh in range(HPT)]

        def wait_idx(bank):
            pltpu.make_async_copy(idx_hbm.at[pl.ds(0, R)], idx_v[bank],
                                  sem_i[bank]).wait()

        def drain_out(bank):
            for hh in range(HPT):
                pltpu.make_async_copy(out_v[bank][hh],
                                      out_hbm.at[0, pl.ds(0, R)],
                                      sem_o[bank]).wait()

        def gather_vreg(bank, q, off):
            idxv = idx_v[bank][q, pl.ds(off, 16)]
            for hh in range(HPT):
                vals = plsc.load_gather(tab_v, [hvecs[hh], idxv])
                out_v[bank][hh][q, pl.ds(off, 16)] = vals

        def gather_chunk(bank):
            @pl.loop(0, R)
            def _(q):
                @pl.loop(0, NVREG - 1)
                def _(j):
                    gather_vreg(bank, q, 16 * j)

                gather_vreg(bank, q, TAIL_OFF)

        def fire_out(bank, r0):
            for hh in range(HPT):
                pltpu.async_copy(out_v[bank][hh],
                                 out_hbm.at[head_base + hh, pl.ds(r0, R)],
                                 sem_o[bank])

        @pl.loop(0, NCHUNK - 1, step=2)
        def _(ci):
            wait_idx(0)
            pltpu.async_copy(idx_hbm.at[pl.ds(chunk_start(ci + 1), R)],
                             idx_v[1], sem_i[1])

            @pl.when(ci > 0)
            def _():
                drain_out(0)

            gather_chunk(0)
            fire_out(0, chunk_start(ci))

            wait_idx(1)
            pltpu.async_copy(idx_hbm.at[pl.ds(chunk_start(ci + 2), R)],
                             idx_v[0], sem_i[0])

            @pl.when(ci > 0)
            def _():
                drain_out(1)

            gather_chunk(1)
            fire_out(1, chunk_start(ci + 1))

        wait_idx(0)
        drain_out(0)
        gather_chunk(0)
        fire_out(0, chunk_start(NCHUNK - 1))

        @pl.when(t == 0)
        def _():
            pltpu.sync_copy(idx_hbm.at[pl.ds(LAST_ROW, 1)], row_i)

            def last_vreg(off):
                idxv = row_i[0, pl.ds(off, 16)]
                for hh in range(HPT):
                    vals = plsc.load_gather(tab_v, [hvecs[hh], idxv])
                    row_o[hh][0, pl.ds(off, 16)] = vals

            @pl.loop(0, NVREG - 1)
            def _(j):
                last_vreg(16 * j)

            last_vreg(TAIL_OFF)
            for hh in range(HPT):
                pltpu.sync_copy(row_o[hh],
                                out_hbm.at[head_base + hh,
                                           pl.ds(LAST_ROW, 1)])

        drain_out(0)
        drain_out(1)

    return kern(table_t, idx)


def kernel(relative_position_bias_table, relative_position_index):
    table_t = relative_position_bias_table.T
    idx = relative_position_index.astype(jnp.int32)
    return _sc_bias_kernel(table_t, idx)


# speedup vs baseline: 2.0163x; 2.0163x over previous
import dataclasses
import functools

import jax
import jax.numpy as jnp
from jax import lax
from jax.experimental import pallas as pl
from jax.experimental.pallas import tpu as pltpu
from jax.experimental.pallas import tpu_sc as plsc

N = 1569
NUM_REL = 10938
NH = 16
HPT = 2
NGRP = NH // HPT
ROWS_PER_TILE = 392
R = 8
NCHUNK = 49
NVREG = 99
TAIL_OFF = N - 16
LAST_ROW = N - 1


def _sc_bias_kernel(table_t, idx):
    mesh = plsc.VectorSubcoreMesh(core_axis_name="c", subcore_axis_name="s")

    blk_f32 = pltpu.VMEM((R, N), jnp.float32)
    blk_i32 = pltpu.VMEM((R, N), jnp.int32)
    row_f32 = pltpu.VMEM((1, N), jnp.float32)

    cp = pltpu.CompilerParams()
    if "needs_layout_passes" in pltpu.CompilerParams.__dataclass_fields__:
        cp = dataclasses.replace(cp, needs_layout_passes=False)

    @functools.partial(
        pl.kernel,
        out_type=jax.ShapeDtypeStruct((NH, N, N), jnp.float32),
        mesh=mesh,
        scratch_types=[
            pltpu.VMEM((HPT, NUM_REL), jnp.float32),
            (blk_i32, blk_i32),
            (tuple(blk_f32 for _ in range(HPT)),
             tuple(blk_f32 for _ in range(HPT))),
            pltpu.VMEM((1, N), jnp.int32),
            tuple(row_f32 for _ in range(HPT)),
            (pltpu.SemaphoreType.DMA, pltpu.SemaphoreType.DMA),
            (pltpu.SemaphoreType.DMA, pltpu.SemaphoreType.DMA),
        ],
        compiler_params=cp,
    )
    def kern(tab_hbm, idx_hbm, out_hbm, tab_v, idx_v, out_v, row_i, row_o,
             sem_i, sem_o):
        c = lax.axis_index("c")
        s = lax.axis_index("s")
        g = s % NGRP
        t = 2 * (s // NGRP) + c
        head_base = HPT * g
        row_base = t * ROWS_PER_TILE

        pltpu.sync_copy(tab_hbm.at[pl.ds(head_base, HPT)], tab_v)

        def chunk_start(ci):
            return pl.multiple_of(row_base + ci * R, R)

        pltpu.async_copy(idx_hbm.at[pl.ds(chunk_start(0), R)],
                         idx_v[0], sem_i[0])

        hvecs = [jnp.full((16,), hh, dtype=jnp.int32) for hh in range(HPT)]

        def wait_idx(bank):
            pltpu.make_async_copy(idx_hbm.at[pl.ds(0, R)], idx_v[bank],
                                  sem_i[bank]).wait()

        def drain_out(bank):
            for hh in range(HPT):
                pltpu.make_async_copy(out_v[bank][hh],
                                      out_hbm.at[0, pl.ds(0, R)],
                                      sem_o[bank]).wait()

        def gather_vreg(bank, q, off):
            idxv = idx_v[bank][q, pl.ds(off, 16)]
            for hh in range(HPT):
                vals = plsc.load_gather(tab_v, [hvecs[hh], idxv])
                out_v[bank][hh][q, pl.ds(off, 16)] = vals

        def gather_chunk(bank):
            @pl.loop(0, R)
            def _(q):
                @plsc.parallel_loop(0, NVREG - 1, unroll=2)
                def _(j):
                    gather_vreg(bank, q, 16 * j)

                gather_vreg(bank, q, TAIL_OFF)

        def fire_out(bank, r0):
            for hh in range(HPT):
                pltpu.async_copy(out_v[bank][hh],
                                 out_hbm.at[head_base + hh, pl.ds(r0, R)],
                                 sem_o[bank])

        @pl.loop(0, NCHUNK - 1, step=2)
        def _(ci):
            wait_idx(0)
            pltpu.async_copy(idx_hbm.at[pl.ds(chunk_start(ci + 1), R)],
                             idx_v[1], sem_i[1])

            @pl.when(ci > 0)
            def _():
                drain_out(0)

            gather_chunk(0)
            fire_out(0, chunk_start(ci))

            wait_idx(1)
            pltpu.async_copy(idx_hbm.at[pl.ds(chunk_start(ci + 2), R)],
                             idx_v[0], sem_i[0])

            @pl.when(ci > 0)
            def _():
                drain_out(1)

            gather_chunk(1)
            fire_out(1, chunk_start(ci + 1))

        wait_idx(0)
        drain_out(0)
        gather_chunk(0)
        fire_out(0, chunk_start(NCHUNK - 1))

        @pl.when(t == 0)
        def _():
            pltpu.sync_copy(idx_hbm.at[pl.ds(LAST_ROW, 1)], row_i)

            def last_vreg(off):
                idxv = row_i[0, pl.ds(off, 16)]
                for hh in range(HPT):
                    vals = plsc.load_gather(tab_v, [hvecs[hh], idxv])
                    row_o[hh][0, pl.ds(off, 16)] = vals

            @pl.loop(0, NVREG - 1)
            def _(j):
                last_vreg(16 * j)

            last_vreg(TAIL_OFF)
            for hh in range(HPT):
                pltpu.sync_copy(row_o[hh],
                                out_hbm.at[head_base + hh,
                                           pl.ds(LAST_ROW, 1)])

        drain_out(0)
        drain_out(1)

    return kern(table_t, idx)


def kernel(relative_position_bias_table, relative_position_index):
    table_t = relative_position_bias_table.T
    idx = relative_position_index.astype(jnp.int32)
    return _sc_bias_kernel(table_t, idx)


# speedup vs baseline: 2.0544x; 1.0189x over previous
import dataclasses
import functools

import jax
import jax.numpy as jnp
from jax import lax
from jax.experimental import pallas as pl
from jax.experimental.pallas import tpu as pltpu
from jax.experimental.pallas import tpu_sc as plsc

N = 1569
NUM_REL = 10938
NH = 16
HPT = 2
NGRP = NH // HPT
ROWS_PER_TILE = 392
R = 8
NCHUNK = 49
NVREG = 99
TAIL_OFF = N - 16
LAST_ROW = N - 1


def _sc_bias_kernel(table_t, idx):
    mesh = plsc.VectorSubcoreMesh(core_axis_name="c", subcore_axis_name="s")

    blk_f32 = pltpu.VMEM((R, N), jnp.float32)
    blk_i32 = pltpu.VMEM((R, N), jnp.int32)
    row_f32 = pltpu.VMEM((1, N), jnp.float32)

    cp = pltpu.CompilerParams()
    if "needs_layout_passes" in pltpu.CompilerParams.__dataclass_fields__:
        cp = dataclasses.replace(cp, needs_layout_passes=False)

    @functools.partial(
        pl.kernel,
        out_type=jax.ShapeDtypeStruct((NH, N, N), jnp.float32),
        mesh=mesh,
        scratch_types=[
            pltpu.VMEM((HPT, NUM_REL), jnp.float32),
            (blk_i32, blk_i32),
            (tuple(blk_f32 for _ in range(HPT)),
             tuple(blk_f32 for _ in range(HPT))),
            pltpu.VMEM((1, N), jnp.int32),
            tuple(row_f32 for _ in range(HPT)),
            (pltpu.SemaphoreType.DMA, pltpu.SemaphoreType.DMA),
            (pltpu.SemaphoreType.DMA, pltpu.SemaphoreType.DMA),
        ],
        compiler_params=cp,
    )
    def kern(tab_hbm, idx_hbm, out_hbm, tab_v, idx_v, out_v, row_i, row_o,
             sem_i, sem_o):
        c = lax.axis_index("c")
        s = lax.axis_index("s")
        g = s % NGRP
        t = 2 * (s // NGRP) + c
        head_base = HPT * g
        row_base = t * ROWS_PER_TILE

        pltpu.sync_copy(tab_hbm.at[pl.ds(head_base, HPT)], tab_v)

        def chunk_start(ci):
            return pl.multiple_of(row_base + ci * R, R)

        pltpu.async_copy(idx_hbm.at[pl.ds(chunk_start(0), R)],
                         idx_v[0], sem_i[0])

        hvecs = [jnp.full((16,), hh, dtype=jnp.int32) for hh in range(HPT)]

        def wait_idx(bank):
            pltpu.make_async_copy(idx_hbm.at[pl.ds(0, R)], idx_v[bank],
                                  sem_i[bank]).wait()

        def drain_out(bank):
            for hh in range(HPT):
                pltpu.make_async_copy(out_v[bank][hh],
                                      out_hbm.at[0, pl.ds(0, R)],
                                      sem_o[bank]).wait()

        def gather_vreg(bank, q, off):
            idxv = idx_v[bank][q, pl.ds(off, 16)]
            for hh in range(HPT):
                vals = plsc.load_gather(tab_v, [hvecs[hh], idxv])
                out_v[bank][hh][q, pl.ds(off, 16)] = vals

        def gather_chunk(bank):
            @pl.loop(0, R)
            def _(q):
                @plsc.parallel_loop(0, NVREG - 1, unroll=4)
                def _(j):
                    gather_vreg(bank, q, 16 * j)

                gather_vreg(bank, q, TAIL_OFF)

        def fire_out(bank, r0):
            for hh in range(HPT):
                pltpu.async_copy(out_v[bank][hh],
                                 out_hbm.at[head_base + hh, pl.ds(r0, R)],
                                 sem_o[bank])

        @pl.loop(0, NCHUNK - 1, step=2)
        def _(ci):
            wait_idx(0)
            pltpu.async_copy(idx_hbm.at[pl.ds(chunk_start(ci + 1), R)],
                             idx_v[1], sem_i[1])

            @pl.when(ci > 0)
            def _():
                drain_out(0)

            gather_chunk(0)
            fire_out(0, chunk_start(ci))

            wait_idx(1)
            pltpu.async_copy(idx_hbm.at[pl.ds(chunk_start(ci + 2), R)],
                             idx_v[0], sem_i[0])

            @pl.when(ci > 0)
            def _():
                drain_out(1)

            gather_chunk(1)
            fire_out(1, chunk_start(ci + 1))

        wait_idx(0)
        drain_out(0)
        gather_chunk(0)
        fire_out(0, chunk_start(NCHUNK - 1))

        @pl.when(t == 0)
        def _():
            pltpu.sync_copy(idx_hbm.at[pl.ds(LAST_ROW, 1)], row_i)

            def last_vreg(off):
                idxv = row_i[0, pl.ds(off, 16)]
                for hh in range(HPT):
                    vals = plsc.load_gather(tab_v, [hvecs[hh], idxv])
                    row_o[hh][0, pl.ds(off, 16)] = vals

            @pl.loop(0, NVREG - 1)
            def _(j):
                last_vreg(16 * j)

            last_vreg(TAIL_OFF)
            for hh in range(HPT):
                pltpu.sync_copy(row_o[hh],
                                out_hbm.at[head_base + hh,
                                           pl.ds(LAST_ROW, 1)])

        drain_out(0)
        drain_out(1)

    return kern(table_t, idx)


def kernel(relative_position_bias_table, relative_position_index):
    table_t = relative_position_bias_table.T
    idx = relative_position_index.astype(jnp.int32)
    return _sc_bias_kernel(table_t, idx)


# speedup vs baseline: 2.0649x; 1.0051x over previous
import dataclasses
import functools

import jax
import jax.numpy as jnp
from jax import lax
from jax.experimental import pallas as pl
from jax.experimental.pallas import tpu as pltpu
from jax.experimental.pallas import tpu_sc as plsc

N = 1569
NUM_REL = 10938
NH = 16
HPT = 2
NGRP = NH // HPT
ROWS_PER_TILE = 392
R = 8
NCHUNK = 49
NVREG = 99
TAIL_OFF = N - 16
LAST_ROW = N - 1


def _sc_bias_kernel(table_t, idx):
    mesh = plsc.VectorSubcoreMesh(core_axis_name="c", subcore_axis_name="s")

    blk_f32 = pltpu.VMEM((R, N), jnp.float32)
    blk_i32 = pltpu.VMEM((R, N), jnp.int32)
    row_f32 = pltpu.VMEM((1, N), jnp.float32)

    cp = pltpu.CompilerParams()
    if "needs_layout_passes" in pltpu.CompilerParams.__dataclass_fields__:
        cp = dataclasses.replace(cp, needs_layout_passes=False)

    @functools.partial(
        pl.kernel,
        out_type=jax.ShapeDtypeStruct((NH, N, N), jnp.float32),
        mesh=mesh,
        scratch_types=[
            pltpu.VMEM((HPT, NUM_REL), jnp.float32),
            (blk_i32, blk_i32),
            (tuple(blk_f32 for _ in range(HPT)),
             tuple(blk_f32 for _ in range(HPT))),
            pltpu.VMEM((1, N), jnp.int32),
            tuple(row_f32 for _ in range(HPT)),
            (pltpu.SemaphoreType.DMA, pltpu.SemaphoreType.DMA),
            (pltpu.SemaphoreType.DMA, pltpu.SemaphoreType.DMA),
        ],
        compiler_params=cp,
    )
    def kern(tab_hbm, idx_hbm, out_hbm, tab_v, idx_v, out_v, row_i, row_o,
             sem_i, sem_o):
        c = lax.axis_index("c")
        s = lax.axis_index("s")
        g = s % NGRP
        t = 2 * (s // NGRP) + c
        head_base = HPT * g
        row_base = t * ROWS_PER_TILE

        pltpu.sync_copy(tab_hbm.at[pl.ds(head_base, HPT)], tab_v)

        def chunk_start(ci):
            return pl.multiple_of(row_base + ci * R, R)

        pltpu.async_copy(idx_hbm.at[pl.ds(chunk_start(0), R)],
                         idx_v[0], sem_i[0])

        hvecs = [jnp.full((16,), hh, dtype=jnp.int32) for hh in range(HPT)]

        def wait_idx(bank):
            pltpu.make_async_copy(idx_hbm.at[pl.ds(0, R)], idx_v[bank],
                                  sem_i[bank]).wait()

        def drain_out(bank):
            for hh in range(HPT):
                pltpu.make_async_copy(out_v[bank][hh],
                                      out_hbm.at[0, pl.ds(0, R)],
                                      sem_o[bank]).wait()

        def gather_vreg(bank, q, off):
            idxv = idx_v[bank][q, pl.ds(off, 16)]
            for hh in range(HPT):
                vals = plsc.load_gather(tab_v, [hvecs[hh], idxv])
                out_v[bank][hh][q, pl.ds(off, 16)] = vals

        def gather_chunk(bank):
            @plsc.parallel_loop(0, NVREG - 1, unroll=2)
            def _(j):
                off = 16 * j
                for q in range(R):
                    gather_vreg(bank, q, off)

            for q in range(R):
                gather_vreg(bank, q, TAIL_OFF)

        def fire_out(bank, r0):
            for hh in range(HPT):
                pltpu.async_copy(out_v[bank][hh],
                                 out_hbm.at[head_base + hh, pl.ds(r0, R)],
                                 sem_o[bank])

        @pl.loop(0, NCHUNK - 1, step=2)
        def _(ci):
            wait_idx(0)
            pltpu.async_copy(idx_hbm.at[pl.ds(chunk_start(ci + 1), R)],
                             idx_v[1], sem_i[1])

            @pl.when(ci > 0)
            def _():
                drain_out(0)

            gather_chunk(0)
            fire_out(0, chunk_start(ci))

            wait_idx(1)
            pltpu.async_copy(idx_hbm.at[pl.ds(chunk_start(ci + 2), R)],
                             idx_v[0], sem_i[0])

            @pl.when(ci > 0)
            def _():
                drain_out(1)

            gather_chunk(1)
            fire_out(1, chunk_start(ci + 1))

        wait_idx(0)
        drain_out(0)
        gather_chunk(0)
        fire_out(0, chunk_start(NCHUNK - 1))

        @pl.when(t == 0)
        def _():
            pltpu.sync_copy(idx_hbm.at[pl.ds(LAST_ROW, 1)], row_i)

            def last_vreg(off):
                idxv = row_i[0, pl.ds(off, 16)]
                for hh in range(HPT):
                    vals = plsc.load_gather(tab_v, [hvecs[hh], idxv])
                    row_o[hh][0, pl.ds(off, 16)] = vals

            @pl.loop(0, NVREG - 1)
            def _(j):
                last_vreg(16 * j)

            last_vreg(TAIL_OFF)
            for hh in range(HPT):
                pltpu.sync_copy(row_o[hh],
                                out_hbm.at[head_base + hh,
                                           pl.ds(LAST_ROW, 1)])

        drain_out(0)
        drain_out(1)

    return kern(table_t, idx)


def kernel(relative_position_bias_table, relative_position_index):
    table_t = relative_position_bias_table.T
    idx = relative_position_index.astype(jnp.int32)
    return _sc_bias_kernel(table_t, idx)
